# baseline (speedup 1.0000x reference)
import jax
import jax.numpy as jnp
from jax import lax
from jax.experimental import pallas as pl
from jax.experimental.pallas import tpu as pltpu
from jax.experimental.pallas import tpu_sc as plsc

_MAX_LEN = 100000
_TIME_DIM = 128
_BATCH = 16384

_NC = 2
_NS = 16
_NW = _NC * _NS
_BPW = _BATCH // _NW
_L = 16


_NCHUNK = 4
_CW = _BPW // _NCHUNK


def _gather_body(t_hbm, pe_hbm, out_hbm, idx_v, rows_v,
                 sg0, sg1, sw0, sw1, sw2, sw3):
    wid = lax.axis_index("s") * _NC + lax.axis_index("c")
    base = wid * _BPW
    pltpu.sync_copy(t_hbm.at[pl.ds(base, _BPW)], idx_v)
    for i in range(_BPW // _L):
        v = idx_v[pl.ds(i * _L, _L)] - 1
        idx_v[pl.ds(i * _L, _L)] = jnp.where(v < 0, v + _MAX_LEN, v)
    gsems = [sg0, sg1]
    wsems = [sw0, sw1, sw2, sw3]
    gathers = [None] * _NCHUNK
    writes = [None] * _NCHUNK

    def start_gather(j):
        return pltpu.async_copy(
            pe_hbm.at[idx_v.at[pl.ds(j * _CW, _CW)]],
            rows_v.at[pl.ds(j * _CW, _CW)],
            gsems[j % 2],
        )

    gathers[0] = start_gather(0)
    for j in range(_NCHUNK):
        if j + 1 < _NCHUNK:
            gathers[j + 1] = start_gather(j + 1)
        gathers[j].wait()
        writes[j] = pltpu.async_copy(
            rows_v.at[pl.ds(j * _CW, _CW)],
            out_hbm.at[pl.ds(base + j * _CW, _CW)],
            wsems[j],
        )
    for j in range(_NCHUNK):
        writes[j].wait()


def kernel(t, pe):
    t32 = t.astype(jnp.int32)
    mesh = plsc.VectorSubcoreMesh(core_axis_name="c", subcore_axis_name="s")
    f = pl.kernel(
        _gather_body,
        mesh=mesh,
        out_type=jax.ShapeDtypeStruct((_BATCH, _TIME_DIM), jnp.float32),
        scratch_types=[
            pltpu.VMEM((_BPW,), jnp.int32),
            pltpu.VMEM((_BPW, _TIME_DIM), jnp.float32),
        ] + [pltpu.SemaphoreType.DMA] * 6,
    )
    return f(t32, pe)


# speedup vs baseline: 1.0178x; 1.0178x over previous
import jax
import jax.numpy as jnp
from jax import lax
from jax.experimental import pallas as pl
from jax.experimental.pallas import tpu as pltpu
from jax.experimental.pallas import tpu_sc as plsc

_MAX_LEN = 100000
_TIME_DIM = 128
_BATCH = 16384

_NC = 2
_NS = 16
_NW = _NC * _NS
_BPW = _BATCH // _NW
_L = 16


_NCHUNK = 8
_CW = _BPW // _NCHUNK


def _gather_body(t_hbm, pe_hbm, out_hbm, idx_v, rows_v, *sems):
    gsems = sems[:_NCHUNK]
    wsems = sems[_NCHUNK:]
    wid = lax.axis_index("s") * _NC + lax.axis_index("c")
    base = wid * _BPW
    pltpu.sync_copy(t_hbm.at[pl.ds(base, _BPW)], idx_v)
    gathers = [None] * _NCHUNK
    writes = [None] * _NCHUNK
    for j in range(_NCHUNK):
        for i in range(j * _CW // _L, (j + 1) * _CW // _L):
            v = idx_v[pl.ds(i * _L, _L)] - 1
            idx_v[pl.ds(i * _L, _L)] = jnp.where(v < 0, v + _MAX_LEN, v)
        gathers[j] = pltpu.async_copy(
            pe_hbm.at[idx_v.at[pl.ds(j * _CW, _CW)]],
            rows_v.at[pl.ds(j * _CW, _CW)],
            gsems[j],
        )
    for j in range(_NCHUNK):
        gathers[j].wait()
        writes[j] = pltpu.async_copy(
            rows_v.at[pl.ds(j * _CW, _CW)],
            out_hbm.at[pl.ds(base + j * _CW, _CW)],
            wsems[j],
        )
    for j in range(_NCHUNK):
        writes[j].wait()


def kernel(t, pe):
    t32 = t.astype(jnp.int32)
    mesh = plsc.VectorSubcoreMesh(core_axis_name="c", subcore_axis_name="s")
    f = pl.kernel(
        _gather_body,
        mesh=mesh,
        out_type=jax.ShapeDtypeStruct((_BATCH, _TIME_DIM), jnp.float32),
        scratch_types=[
            pltpu.VMEM((_BPW,), jnp.int32),
            pltpu.VMEM((_BPW, _TIME_DIM), jnp.float32),
        ] + [pltpu.SemaphoreType.DMA] * (2 * _NCHUNK),
    )
    return f(t32, pe)


# speedup vs baseline: 1.0419x; 1.0237x over previous
import jax
import jax.numpy as jnp
from jax import lax
from jax.experimental import pallas as pl
from jax.experimental.pallas import tpu as pltpu
from jax.experimental.pallas import tpu_sc as plsc

_MAX_LEN = 100000
_TIME_DIM = 128
_BATCH = 16384

_NC = 2
_NS = 16
_NW = _NC * _NS
_BPW = _BATCH // _NW
_L = 16


def _gather_body(t_hbm, pe_hbm, out_hbm, idx_v, rows_v, sem):
    wid = lax.axis_index("s") * _NC + lax.axis_index("c")
    base = wid * _BPW
    pltpu.sync_copy(t_hbm.at[pl.ds(base, _BPW)], idx_v)

    def fix(i, carry):
        s = i * _L
        v = idx_v[pl.ds(s, _L)] - 1
        idx_v[pl.ds(s, _L)] = jnp.where(v < 0, v + _MAX_LEN, v)
        return carry

    lax.fori_loop(0, _BPW // _L, fix, 0)
    pltpu.async_copy(pe_hbm.at[idx_v], rows_v, sem).wait()
    pltpu.sync_copy(rows_v, out_hbm.at[pl.ds(base, _BPW)])


def kernel(t, pe):
    t32 = t.astype(jnp.int32)
    mesh = plsc.VectorSubcoreMesh(core_axis_name="c", subcore_axis_name="s")
    f = pl.kernel(
        _gather_body,
        mesh=mesh,
        out_type=jax.ShapeDtypeStruct((_BATCH, _TIME_DIM), jnp.float32),
        scratch_types=[
            pltpu.VMEM((_BPW,), jnp.int32),
            pltpu.VMEM((_BPW, _TIME_DIM), jnp.float32),
            pltpu.SemaphoreType.DMA,
        ],
    )
    return f(t32, pe)
